# baseline (speedup 1.0000x reference)
import functools

import jax
import jax.numpy as jnp
from jax import lax
from jax.experimental import pallas as pl
from jax.experimental.pallas import tpu as pltpu
from jax.experimental.pallas import tpu_sc as plsc

_SPLIT = 4
_REP = 8
_NBUF = 8
_DEPTH = _NBUF // 2


def kernel(prefix, table):
    B, P = prefix.shape
    V, D = table.shape
    N = B * P
    Dq = D // _SPLIT
    NR = N * _SPLIT

    flat = prefix.reshape(N).astype(jnp.int32)
    idx = jnp.repeat(
        (_SPLIT * flat[:, None] + jnp.arange(_SPLIT, dtype=jnp.int32)).reshape(-1),
        _REP,
    )
    tableq = table.reshape(V * _SPLIT, Dq)

    info = plsc.get_sparse_core_info()
    NC, NS = info.num_cores, info.num_subcores
    NW = NC * NS
    n_per_w = NR // NW

    mesh = plsc.VectorSubcoreMesh(core_axis_name="c", subcore_axis_name="s")

    scratch = (
        [pltpu.VMEM((n_per_w * _REP,), jnp.int32)]
        + [pltpu.VMEM((1, Dq), jnp.float32) for _ in range(_NBUF)]
        + [pltpu.SemaphoreType.DMA for _ in range(2 * _NBUF)]
    )

    @functools.partial(
        pl.kernel,
        out_type=jax.ShapeDtypeStruct((NR, Dq), jnp.float32),
        mesh=mesh,
        scratch_types=scratch,
    )
    def gather_kernel(idx_hbm, table_hbm, out_hbm, idx_v, *rest):
        bufs = rest[:_NBUF]
        gsems = rest[_NBUF : 2 * _NBUF]
        wsems = rest[2 * _NBUF :]
        wid = lax.axis_index("s") * NC + lax.axis_index("c")
        base = wid * n_per_w
        pltpu.sync_copy(idx_hbm.at[pl.ds(base * _REP, n_per_w * _REP)], idx_v)

        def gather(s, j):
            off = pl.multiple_of(s * _REP, _REP)
            pltpu.async_copy(
                table_hbm.at[idx_v.at[pl.ds(off, 1)]], bufs[j], gsems[j]
            )

        def wait_gather(j):
            pltpu.make_async_copy(
                table_hbm.at[pl.ds(0, 1)], bufs[j], gsems[j]
            ).wait()

        def write(s, j):
            pltpu.async_copy(bufs[j], out_hbm.at[pl.ds(base + s, 1)], wsems[j])

        def wait_write(j):
            pltpu.make_async_copy(
                bufs[j], out_hbm.at[pl.ds(base, 1)], wsems[j]
            ).wait()

        for s in range(_DEPTH):
            gather(s, s)
        for s in range(_DEPTH):
            wait_gather(s)
            write(s, s)
            gather(s + _DEPTH, (s + _DEPTH) % _NBUF)

        def body(q, carry):
            for j in range(_NBUF):
                s = _NBUF * q + _DEPTH + j
                b = (_DEPTH + j) % _NBUF
                wait_gather(b)
                write(s, b)
                wait_write(j)
                gather(s + _DEPTH, j)
            return carry

        n_steady = (n_per_w - 2 * _DEPTH) // _NBUF
        lax.fori_loop(0, n_steady, body, 0, unroll=False)

        for j in range(_DEPTH):
            s = n_per_w - _DEPTH + j
            b = s % _NBUF
            wait_gather(b)
            write(s, b)
            wait_write(j)
        for j in range(_DEPTH):
            wait_write((_DEPTH + j) % _NBUF)

    out = gather_kernel(idx, tableq)
    return out.reshape(B, P, D)


# speedup vs baseline: 2.2919x; 2.2919x over previous
import functools

import jax
import jax.numpy as jnp
from jax import lax
from jax.experimental import pallas as pl
from jax.experimental.pallas import tpu as pltpu
from jax.experimental.pallas import tpu_sc as plsc


def kernel(prefix, table):
    B, P = prefix.shape
    V, D = table.shape
    N = B * P

    idx = jnp.repeat(prefix.reshape(N).astype(jnp.int32), 8)

    info = plsc.get_sparse_core_info()
    NC, NS = info.num_cores, info.num_subcores
    NW = NC * NS
    n_per_w = N // NW

    mesh = plsc.VectorSubcoreMesh(core_axis_name="c", subcore_axis_name="s")

    @functools.partial(
        pl.kernel,
        out_type=jax.ShapeDtypeStruct((N, D), jnp.float32),
        mesh=mesh,
        scratch_types=[
            pltpu.VMEM((n_per_w * 8,), jnp.int32),
            pltpu.VMEM((1, D), jnp.float32),
            pltpu.VMEM((1, D), jnp.float32),
            pltpu.SemaphoreType.DMA,
            pltpu.SemaphoreType.DMA,
            pltpu.SemaphoreType.DMA,
            pltpu.SemaphoreType.DMA,
        ],
    )
    def gather_kernel(
        idx_hbm, table_hbm, out_hbm, idx_v, buf0, buf1, gs0, gs1, ws0, ws1
    ):
        wid = lax.axis_index("s") * NC + lax.axis_index("c")
        base = wid * n_per_w
        pltpu.sync_copy(idx_hbm.at[pl.ds(base * 8, n_per_w * 8)], idx_v)

        def gather(g, buf, sem):
            off = pl.multiple_of(g * 8, 8)
            pltpu.async_copy(table_hbm.at[idx_v.at[pl.ds(off, 1)]], buf, sem)

        def write(g, buf, sem):
            pltpu.async_copy(buf, out_hbm.at[pl.ds(base + g, 1)], sem)

        gather(0, buf0, gs0)

        def pair(p, carry):
            g = 2 * p
            pltpu.make_async_copy(table_hbm.at[pl.ds(0, 1)], buf0, gs0).wait()
            write(g, buf0, ws0)

            @pl.when(p > 0)
            def _():
                pltpu.make_async_copy(
                    buf1, out_hbm.at[pl.ds(base, 1)], ws1
                ).wait()

            gather(g + 1, buf1, gs1)
            pltpu.make_async_copy(table_hbm.at[pl.ds(0, 1)], buf1, gs1).wait()
            write(g + 1, buf1, ws1)
            pltpu.make_async_copy(buf0, out_hbm.at[pl.ds(base, 1)], ws0).wait()

            @pl.when(p < n_per_w // 2 - 1)
            def _():
                gather(g + 2, buf0, gs0)

            return carry

        lax.fori_loop(0, n_per_w // 2, pair, 0)
        pltpu.make_async_copy(buf1, out_hbm.at[pl.ds(base, 1)], ws1).wait()

    out = gather_kernel(idx, table)
    return out.reshape(B, P, D)


# speedup vs baseline: 2.2973x; 1.0023x over previous
import functools

import jax
import jax.numpy as jnp
from jax import lax
from jax.experimental import pallas as pl
from jax.experimental.pallas import tpu as pltpu
from jax.experimental.pallas import tpu_sc as plsc


def kernel(prefix, table):
    B, P = prefix.shape
    V, D = table.shape
    N = B * P

    idx = jnp.repeat(prefix.reshape(N).astype(jnp.int32), 8)

    info = plsc.get_sparse_core_info()
    NC, NS = info.num_cores, info.num_subcores
    NW = NC * NS
    n_per_w = N // NW
    n_pair = n_per_w // 2

    mesh = plsc.VectorSubcoreMesh(core_axis_name="c", subcore_axis_name="s")

    @functools.partial(
        pl.kernel,
        out_type=jax.ShapeDtypeStruct((N, D), jnp.float32),
        mesh=mesh,
        scratch_types=[
            pltpu.VMEM((n_per_w * 8,), jnp.int32),
            pltpu.VMEM((1, D), jnp.float32),
            pltpu.VMEM((1, D), jnp.float32),
            pltpu.SemaphoreType.DMA,
            pltpu.SemaphoreType.DMA,
            pltpu.SemaphoreType.DMA,
            pltpu.SemaphoreType.DMA,
        ],
    )
    def gather_kernel(
        idx_hbm, table_hbm, out_hbm, idx_v, buf0, buf1, gs0, gs1, ws0, ws1
    ):
        wid = lax.axis_index("s") * NC + lax.axis_index("c")
        base = wid * n_per_w
        pltpu.sync_copy(idx_hbm.at[pl.ds(base * 8, n_per_w * 8)], idx_v)

        def gather(g, buf, sem):
            off = pl.multiple_of(g * 8, 8)
            pltpu.async_copy(table_hbm.at[idx_v.at[pl.ds(off, 1)]], buf, sem)

        def wait_gather(buf, sem):
            pltpu.make_async_copy(table_hbm.at[pl.ds(0, 1)], buf, sem).wait()

        def write(g, buf, sem):
            pltpu.async_copy(buf, out_hbm.at[pl.ds(base + g, 1)], sem)

        def wait_write(buf, sem):
            pltpu.make_async_copy(buf, out_hbm.at[pl.ds(base, 1)], sem).wait()

        gather(0, buf0, gs0)
        gather(1, buf1, gs1)
        wait_gather(buf0, gs0)
        write(0, buf0, ws0)
        wait_gather(buf1, gs1)
        write(1, buf1, ws1)
        wait_write(buf0, ws0)
        gather(2, buf0, gs0)

        def pair(p, carry):
            g = 2 * p
            wait_gather(buf0, gs0)
            write(g, buf0, ws0)
            wait_write(buf1, ws1)
            gather(g + 1, buf1, gs1)
            wait_gather(buf1, gs1)
            write(g + 1, buf1, ws1)
            wait_write(buf0, ws0)
            gather(g + 2, buf0, gs0)
            return carry

        lax.fori_loop(1, n_pair - 1, pair, 0)

        g = n_per_w - 2
        wait_gather(buf0, gs0)
        write(g, buf0, ws0)
        wait_write(buf1, ws1)
        gather(g + 1, buf1, gs1)
        wait_gather(buf1, gs1)
        write(g + 1, buf1, ws1)
        wait_write(buf0, ws0)
        wait_write(buf1, ws1)

    out = gather_kernel(idx, table)
    return out.reshape(B, P, D)
